# Initial kernel scaffold; baseline (speedup 1.0000x reference)
import jax
import jax.numpy as jnp
from jax.experimental import pallas as pl


def kernel(points, indices):
    raise NotImplementedError("write your pallas kernel here")


# speedup vs baseline: 2.2087x; 2.2087x over previous
import functools

import jax
import jax.numpy as jnp
from jax import lax
from jax.experimental import pallas as pl
from jax.experimental.pallas import tpu as pltpu
from jax.experimental.pallas import tpu_sc as plsc

L = 16


@functools.lru_cache(maxsize=None)
def _build(M, K, D, NC, NS):
    NW = NC * NS
    B = 4
    S = M // B
    NITER = (S + NW - 1) // NW
    mesh = plsc.VectorSubcoreMesh(core_axis_name="c", subcore_axis_name="s")

    @functools.partial(
        pl.kernel,
        out_type=jax.ShapeDtypeStruct((M, D), jnp.float32),
        mesh=mesh,
        scratch_types=[
            pltpu.VMEM((B * K,), jnp.int32),
            pltpu.VMEM((B * K, D), jnp.float32),
            pltpu.VMEM((B, D), jnp.float32),
            pltpu.SemaphoreType.DMA,
        ],
    )
    def pool(points_hbm, idx_hbm, out_hbm, idx_v, rows_v, out_v, sem):
        wid = lax.axis_index("s") * NC + lax.axis_index("c")
        inv = jnp.float32(1.0 / K)

        def step(i, carry):
            st = wid + i * NW

            @pl.when(st < S)
            def _():
                base = st * B
                pltpu.sync_copy(idx_hbm.at[pl.ds(base * K, B * K)], idx_v)
                pltpu.async_copy(points_hbm.at[idx_v], rows_v, sem).wait()
                for r in range(B):
                    for ch in range(D // L):
                        sl = pl.ds(ch * L, L)
                        acc = rows_v[r * K, sl]
                        for kk in range(1, K):
                            acc = acc + rows_v[r * K + kk, sl]
                        out_v[r, sl] = acc * inv
                pltpu.sync_copy(out_v, out_hbm.at[pl.ds(base, B)])

            return carry

        lax.fori_loop(0, NITER, step, 0)

    return pool


def kernel(points, indices):
    M, D = points.shape
    K = indices.shape[1]
    info = plsc.get_sparse_core_info()
    idx_flat = indices.astype(jnp.int32).reshape(-1)
    return _build(M, K, D, info.num_cores, info.num_subcores)(points, idx_flat)


# speedup vs baseline: 5.7222x; 2.5907x over previous
import functools

import jax
import jax.numpy as jnp
from jax import lax
from jax.experimental import pallas as pl
from jax.experimental.pallas import tpu as pltpu
from jax.experimental.pallas import tpu_sc as plsc

L = 16


@functools.lru_cache(maxsize=None)
def _build(M, K, D, NC, NS):
    NW = NC * NS
    B = 4
    G = B * K
    S = M // B
    NITER = ((S + NW - 1) // NW + 7) // 8 * 8
    NPAD = (NITER + 1) // 2 * 2
    mesh = plsc.VectorSubcoreMesh(core_axis_name="c", subcore_axis_name="s")

    scratch = (
        [pltpu.VMEM((G,), jnp.int32) for _ in range(2)]
        + [pltpu.VMEM((G, D), jnp.float32) for _ in range(2)]
        + [pltpu.VMEM((B, D), jnp.float32)]
        + [pltpu.SemaphoreType.DMA for _ in range(2)]
    )

    @functools.partial(
        pl.kernel,
        out_type=jax.ShapeDtypeStruct((M, D), jnp.float32),
        mesh=mesh,
        scratch_types=scratch,
    )
    def pool(points_hbm, idx_hbm, out_hbm, idxb0, idxb1, rows0, rows1,
             out_v, gsem0, gsem1):
        idxb = (idxb0, idxb1)
        rows = (rows0, rows1)
        gsem = (gsem0, gsem1)
        wid = lax.axis_index("s") * NC + lax.axis_index("c")
        lo = wid * NITER
        inv = jnp.float32(1.0 / K)

        def valid(j):
            return (j < NITER) & (lo + j < S)

        def stage(j, b):
            pltpu.sync_copy(idx_hbm.at[pl.ds((lo + j) * G, G)], idxb[b])

        def gather(j, b):
            return pltpu.make_async_copy(
                points_hbm.at[idxb[b]], rows[b], gsem[b])

        @pl.when(valid(0))
        def _():
            stage(0, 0)
            gather(0, 0).start()

        def step(j, b):
            @pl.when(valid(j + 1))
            def _():
                stage(j + 1, b ^ 1)
                gather(j + 1, b ^ 1).start()

            @pl.when(valid(j))
            def _():
                gather(j, b).wait()
                NCH = D // L
                KU = 4
                for r in range(B):
                    zero = jnp.zeros((L,), jnp.float32)

                    def kbody(t, accs, r=r):
                        base = r * K + t * KU
                        new = []
                        for ch in range(NCH):
                            sl = pl.ds(ch * L, L)
                            a = accs[ch]
                            for u in range(KU):
                                a = a + rows[b][base + u, sl]
                            new.append(a)
                        return tuple(new)

                    accs = lax.fori_loop(0, K // KU, kbody, (zero,) * NCH)
                    for ch in range(NCH):
                        out_v[r, pl.ds(ch * L, L)] = accs[ch] * inv
                pltpu.sync_copy(out_v, out_hbm.at[pl.ds((lo + j) * B, B)])

        @pl.loop(0, NPAD, step=2)
        def _(g):
            for b in range(2):
                step(g + b, b)

    def run(points, idx_flat):
        pad = NW * NITER * G - idx_flat.shape[0]
        idx_padded = jnp.pad(idx_flat, (0, pad)) if pad else idx_flat
        return pool(points, idx_padded)

    return run


def kernel(points, indices):
    M, D = points.shape
    K = indices.shape[1]
    info = plsc.get_sparse_core_info()
    idx_flat = indices.astype(jnp.int32).reshape(-1)
    return _build(M, K, D, info.num_cores, info.num_subcores)(points, idx_flat)


# speedup vs baseline: 5.8472x; 1.0218x over previous
import functools

import jax
import jax.numpy as jnp
from jax import lax
from jax.experimental import pallas as pl
from jax.experimental.pallas import tpu as pltpu
from jax.experimental.pallas import tpu_sc as plsc

L = 16


@functools.lru_cache(maxsize=None)
def _build(M, K, D, NC, NS):
    NW = NC * NS
    B = 4
    G = B * K
    S = M // B
    NITER = ((S + NW - 1) // NW + 7) // 8 * 8
    NPAD = (NITER + 1) // 2 * 2
    mesh = plsc.VectorSubcoreMesh(core_axis_name="c", subcore_axis_name="s")

    scratch = (
        [pltpu.VMEM((G,), jnp.int32) for _ in range(2)]
        + [pltpu.VMEM((G, D), jnp.float32) for _ in range(2)]
        + [pltpu.VMEM((B, D), jnp.float32) for _ in range(2)]
        + [pltpu.SemaphoreType.DMA for _ in range(4)]
    )

    @functools.partial(
        pl.kernel,
        out_type=jax.ShapeDtypeStruct((M, D), jnp.float32),
        mesh=mesh,
        scratch_types=scratch,
    )
    def pool(points_hbm, idx_hbm, out_hbm, idxb0, idxb1, rows0, rows1,
             outv0, outv1, gsem0, gsem1, ssem0, ssem1):
        idxb = (idxb0, idxb1)
        rows = (rows0, rows1)
        outs = (outv0, outv1)
        gsem = (gsem0, gsem1)
        ssem = (ssem0, ssem1)
        wid = lax.axis_index("s") * NC + lax.axis_index("c")
        lo = wid * NITER
        inv = jnp.float32(1.0 / K)

        def valid(j):
            return (j < NITER) & (lo + j < S)

        def stage(j, b):
            pltpu.sync_copy(idx_hbm.at[pl.ds((lo + j) * G, G)], idxb[b])

        def gather(j, b):
            return pltpu.make_async_copy(
                points_hbm.at[idxb[b]], rows[b], gsem[b])

        def store(j, b):
            return pltpu.make_async_copy(
                outs[b], out_hbm.at[pl.ds((lo + j) * B, B)], ssem[b])

        @pl.when(valid(0))
        def _():
            stage(0, 0)
            gather(0, 0).start()

        def step(j, b):
            @pl.when(valid(j + 1))
            def _():
                stage(j + 1, b ^ 1)
                gather(j + 1, b ^ 1).start()

            @pl.when(valid(j))
            def _():
                gather(j, b).wait()

                @pl.when(j >= 2)
                def _():
                    store(j - 2, b).wait()

                NCH = D // L
                KU = 4
                for r in range(B):
                    zero = jnp.zeros((L,), jnp.float32)

                    def kbody(t, accs, r=r):
                        base = r * K + t * KU
                        new = []
                        for ch in range(NCH):
                            sl = pl.ds(ch * L, L)
                            a = accs[ch]
                            for u in range(KU):
                                a = a + rows[b][base + u, sl]
                            new.append(a)
                        return tuple(new)

                    accs = lax.fori_loop(0, K // KU, kbody, (zero,) * NCH)
                    for ch in range(NCH):
                        outs[b][r, pl.ds(ch * L, L)] = accs[ch] * inv
                store(j, b).start()

        @pl.loop(0, NPAD, step=2)
        def _(g):
            for b in range(2):
                step(g + b, b)

        for j in range(NPAD - 2, NPAD):
            @pl.when(valid(j))
            def _(j=j):
                store(j, j % 2).wait()

    def run(points, idx_flat):
        pad = NW * NITER * G - idx_flat.shape[0]
        idx_padded = jnp.pad(idx_flat, (0, pad)) if pad else idx_flat
        return pool(points, idx_padded)

    return run


def kernel(points, indices):
    M, D = points.shape
    K = indices.shape[1]
    info = plsc.get_sparse_core_info()
    idx_flat = indices.astype(jnp.int32).reshape(-1)
    return _build(M, K, D, info.num_cores, info.num_subcores)(points, idx_flat)


# speedup vs baseline: 6.7147x; 1.1484x over previous
import functools

import jax
import jax.numpy as jnp
from jax import lax
from jax.experimental import pallas as pl
from jax.experimental.pallas import tpu as pltpu
from jax.experimental.pallas import tpu_sc as plsc

L = 16


@functools.lru_cache(maxsize=None)
def _build(M, K, D, NC, NS):
    NW = NC * NS
    B = 4
    G = B * K
    S = M // B
    NITER = ((S + NW - 1) // NW + 7) // 8 * 8
    NPAD = (NITER + 1) // 2 * 2
    mesh = plsc.VectorSubcoreMesh(core_axis_name="c", subcore_axis_name="s")

    scratch = (
        [pltpu.VMEM((G,), jnp.int32) for _ in range(2)]
        + [pltpu.VMEM((G, D), jnp.float32) for _ in range(2)]
        + [pltpu.VMEM((B, D), jnp.float32) for _ in range(2)]
        + [pltpu.SemaphoreType.DMA for _ in range(6)]
    )

    @functools.partial(
        pl.kernel,
        out_type=jax.ShapeDtypeStruct((M, D), jnp.float32),
        mesh=mesh,
        scratch_types=scratch,
    )
    def pool(points_hbm, idx_hbm, out_hbm, idxb0, idxb1, rows0, rows1,
             outv0, outv1, gsem0, gsem1, ssem0, ssem1, isem0, isem1):
        idxb = (idxb0, idxb1)
        rows = (rows0, rows1)
        outs = (outv0, outv1)
        gsem = (gsem0, gsem1)
        ssem = (ssem0, ssem1)
        isem = (isem0, isem1)
        wid = lax.axis_index("s") * NC + lax.axis_index("c")
        lo = wid * NITER
        inv = jnp.float32(1.0 / K)

        def valid(j):
            return (j < NITER) & (lo + j < S)

        def idx_load(j, b):
            return pltpu.make_async_copy(
                idx_hbm.at[pl.ds((lo + j) * G, G)], idxb[b], isem[b])

        def gather(j, b):
            return pltpu.make_async_copy(
                points_hbm.at[idxb[b]], rows[b], gsem[b])

        def store(j, b):
            return pltpu.make_async_copy(
                outs[b], out_hbm.at[pl.ds((lo + j) * B, B)], ssem[b])

        @pl.when(valid(0))
        def _():
            idx_load(0, 0).start()
            idx_load(0, 0).wait()
            gather(0, 0).start()

        @pl.when(valid(1))
        def _():
            idx_load(1, 1).start()

        def step(j, b):
            @pl.when(valid(j + 1))
            def _():
                idx_load(j + 1, b ^ 1).wait()
                gather(j + 1, b ^ 1).start()

            @pl.when(valid(j))
            def _():
                gather(j, b).wait()

                @pl.when(valid(j + 2))
                def _():
                    idx_load(j + 2, b).start()

                @pl.when(j >= 2)
                def _():
                    store(j - 2, b).wait()

                NCH = D // L
                KU = 8
                for r in range(B):
                    zero = jnp.zeros((L,), jnp.float32)

                    def kbody(t, accs, r=r):
                        base = r * K + t * KU
                        new = []
                        for ch in range(NCH):
                            sl = pl.ds(ch * L, L)
                            a = accs[ch]
                            for u in range(KU):
                                a = a + rows[b][base + u, sl]
                            new.append(a)
                        return tuple(new)

                    accs = lax.fori_loop(0, K // KU, kbody, (zero,) * NCH)
                    for ch in range(NCH):
                        outs[b][r, pl.ds(ch * L, L)] = accs[ch] * inv
                store(j, b).start()

        @pl.loop(0, NPAD, step=2)
        def _(g):
            for b in range(2):
                step(g + b, b)

        for j in range(NPAD - 2, NPAD):
            @pl.when(valid(j))
            def _(j=j):
                store(j, j % 2).wait()

    def run(points, idx_flat):
        pad = NW * NITER * G - idx_flat.shape[0]
        idx_padded = jnp.pad(idx_flat, (0, pad)) if pad else idx_flat
        return pool(points, idx_padded)

    return run


def kernel(points, indices):
    M, D = points.shape
    K = indices.shape[1]
    info = plsc.get_sparse_core_info()
    idx_flat = indices.astype(jnp.int32).reshape(-1)
    return _build(M, K, D, info.num_cores, info.num_subcores)(points, idx_flat)


# speedup vs baseline: 6.9935x; 1.0415x over previous
import functools

import jax
import jax.numpy as jnp
from jax import lax
from jax.experimental import pallas as pl
from jax.experimental.pallas import tpu as pltpu
from jax.experimental.pallas import tpu_sc as plsc

L = 16


@functools.lru_cache(maxsize=None)
def _build(M, K, D, NC, NS):
    NW = NC * NS
    B = 4
    G = B * K
    S = M // B
    NITER = ((S + NW - 1) // NW + 7) // 8 * 8
    NBUF = 3
    NPAD = (NITER + NBUF - 1) // NBUF * NBUF
    mesh = plsc.VectorSubcoreMesh(core_axis_name="c", subcore_axis_name="s")

    scratch = (
        [pltpu.VMEM((G,), jnp.int32) for _ in range(NBUF)]
        + [pltpu.VMEM((G, D), jnp.float32) for _ in range(NBUF)]
        + [pltpu.VMEM((B, D), jnp.float32) for _ in range(NBUF)]
        + [pltpu.SemaphoreType.DMA for _ in range(3 * NBUF)]
    )

    @functools.partial(
        pl.kernel,
        out_type=jax.ShapeDtypeStruct((M, D), jnp.float32),
        mesh=mesh,
        scratch_types=scratch,
    )
    def pool(points_hbm, idx_hbm, out_hbm, *bufs):
        idxb = bufs[0:NBUF]
        rows = bufs[NBUF:2 * NBUF]
        outs = bufs[2 * NBUF:3 * NBUF]
        isem = bufs[3 * NBUF:4 * NBUF]
        gsem = bufs[4 * NBUF:5 * NBUF]
        ssem = bufs[5 * NBUF:6 * NBUF]
        wid = lax.axis_index("s") * NC + lax.axis_index("c")
        lo = wid * NITER
        n_valid = jnp.minimum(NITER, S - lo)
        inv = jnp.float32(1.0 / K)

        def valid(j):
            return j < n_valid

        def idx_load(j, b):
            return pltpu.make_async_copy(
                idx_hbm.at[pl.ds((lo + j) * G, G)], idxb[b], isem[b])

        def gather(j, b):
            return pltpu.make_async_copy(
                points_hbm.at[idxb[b]], rows[b], gsem[b])

        def store(j, b):
            return pltpu.make_async_copy(
                outs[b], out_hbm.at[pl.ds((lo + j) * B, B)], ssem[b])

        for b in range(NBUF - 1):
            @pl.when(valid(b))
            def _(b=b):
                idx_load(b, b).start()
                idx_load(b, b).wait()
                gather(b, b).start()

        @pl.when(valid(NBUF - 1))
        def _():
            idx_load(NBUF - 1, NBUF - 1).start()

        def step(j, b):
            j2 = j + NBUF - 1
            b2 = (b + NBUF - 1) % NBUF

            @pl.when(valid(j2))
            def _():
                idx_load(j2, b2).wait()
                gather(j2, b2).start()

            @pl.when(valid(j))
            def _():
                gather(j, b).wait()

                @pl.when(valid(j + NBUF))
                def _():
                    idx_load(j + NBUF, b).start()

                @pl.when(j >= NBUF)
                def _():
                    store(j - NBUF, b).wait()

                NCH = D // L
                KU = 8
                for r in range(B):
                    zero = jnp.zeros((L,), jnp.float32)

                    def kbody(t, accs, r=r):
                        base = r * K + t * KU
                        new = []
                        for ch in range(NCH):
                            sl = pl.ds(ch * L, L)
                            a = accs[ch]
                            for u in range(KU):
                                a = a + rows[b][base + u, sl]
                            new.append(a)
                        return tuple(new)

                    accs = lax.fori_loop(0, K // KU, kbody, (zero,) * NCH)
                    for ch in range(NCH):
                        outs[b][r, pl.ds(ch * L, L)] = accs[ch] * inv
                store(j, b).start()

        @pl.loop(0, NPAD, step=NBUF)
        def _(g):
            for b in range(NBUF):
                step(g + b, b)

        for b in range(NBUF):
            @pl.when(n_valid > b)
            def _(b=b):
                store(b, b).wait()

    def run(points, idx_flat):
        pad = NW * NITER * G - idx_flat.shape[0]
        idx_padded = jnp.pad(idx_flat, (0, pad)) if pad else idx_flat
        return pool(points, idx_padded)

    return run


def kernel(points, indices):
    M, D = points.shape
    K = indices.shape[1]
    info = plsc.get_sparse_core_info()
    idx_flat = indices.astype(jnp.int32).reshape(-1)
    return _build(M, K, D, info.num_cores, info.num_subcores)(points, idx_flat)
